# Initial kernel scaffold; baseline (speedup 1.0000x reference)
import jax
import jax.numpy as jnp
from jax.experimental import pallas as pl


def kernel(x, edge_index, edge_attr, batch, W_feat, b_feat, vn_emb, W_lin, b_lin, W_edge, b_edge, root_emb, bn_gamma, bn_beta):
    raise NotImplementedError("write your pallas kernel here")


# speedup vs baseline: 7.3327x; 7.3327x over previous
import functools

import jax
import jax.numpy as jnp
from jax import lax
from jax.experimental import pallas as pl
from jax.experimental.pallas import tpu as pltpu
from jax.experimental.pallas import tpu_sc as plsc

N_ = 10000
E_ = 320000
D_ = 128
DE_ = 16
EPS_ = 1e-5

NC_ = 2
NS_ = 16
NW_ = NC_ * NS_
CE_ = 128
NCH_ = E_ // CE_
NP_ = 10112
RPT_ = 632
RLAST_ = N_ - 15 * RPT_


def _deg_body(row_hbm, out_hbm, cnt_sh, idx_v, ones_v, zb_v, sem):
    c = lax.axis_index("c")
    s = lax.axis_index("s")
    w = s * NC_ + c
    nch = jnp.where(w < NCH_ % NW_, NCH_ // NW_ + 1, NCH_ // NW_)

    for i in range(CE_ // 16):
        ones_v[pl.ds(i * 16, 16)] = jnp.ones((16,), jnp.float32)
    for i in range(640 // 16):
        zb_v[pl.ds(i * 16, 16)] = jnp.zeros((16,), jnp.float32)

    @pl.when(s < 15)
    def _():
        pltpu.sync_copy(zb_v, cnt_sh.at[pl.ds(s * 640, 640)])

    @pl.when(s == 15)
    def _():
        pltpu.sync_copy(zb_v.at[pl.ds(0, 512)], cnt_sh.at[pl.ds(9600, 512)])

    plsc.subcore_barrier()

    def chunk(i, carry):
        base = (w + NW_ * i) * CE_
        pltpu.sync_copy(row_hbm.at[pl.ds(base, CE_)], idx_v)
        pltpu.sync_copy(ones_v, cnt_sh.at[idx_v], add=True)
        return carry

    lax.fori_loop(0, nch, chunk, 0)
    plsc.subcore_barrier()

    @pl.when(s < 15)
    def _():
        pltpu.sync_copy(cnt_sh.at[pl.ds(s * 640, 640)],
                        out_hbm.at[pl.ds(c * NP_ + s * 640, 640)])

    @pl.when(s == 15)
    def _():
        pltpu.sync_copy(cnt_sh.at[pl.ds(9600, 512)],
                        out_hbm.at[pl.ds(c * NP_ + 9600, 512)])


@functools.cache
def _sc_degree():
    mesh = plsc.VectorSubcoreMesh(core_axis_name="c", subcore_axis_name="s",
                                  num_cores=NC_, num_subcores=NS_)
    return pl.kernel(
        _deg_body,
        out_type=jax.ShapeDtypeStruct((NC_ * NP_,), jnp.float32),
        mesh=mesh,
        scratch_types=[
            pltpu.VMEM_SHARED((NP_,), jnp.float32),
            pltpu.VMEM((CE_,), jnp.int32),
            pltpu.VMEM((CE_,), jnp.float32),
            pltpu.VMEM((640,), jnp.float32),
            pltpu.SemaphoreType.DMA,
        ],
    )


def _edge_body(rowi_hbm, coli_hbm, e_hbm, hx_hbm, dinv_hbm, out_hbm,
               agg_sh, row_v, col_v, hxg_v, msg_v, nrm_v, zb_v, sem):
    c = lax.axis_index("c")
    s = lax.axis_index("s")
    w = s * NC_ + c
    nch = jnp.where(w < NCH_ % NW_, NCH_ // NW_ + 1, NCH_ // NW_)

    for i in range(8):
        for j in range(D_ // 16):
            zb_v[i, pl.ds(j * 16, 16)] = jnp.zeros((16,), jnp.float32)

    nz = jnp.where(s < 15, RPT_ // 8, RLAST_ // 8)

    def zloop(i, carry):
        pltpu.sync_copy(zb_v, agg_sh.at[pl.ds(s * RPT_ + i * 8, 8)])
        return carry

    lax.fori_loop(0, nz, zloop, 0)
    plsc.subcore_barrier()

    def chunk(i, carry):
        base = (w + NW_ * i) * CE_
        pltpu.sync_copy(rowi_hbm.at[pl.ds(base, CE_)], row_v)
        pltpu.sync_copy(coli_hbm.at[pl.ds(base, CE_)], col_v)
        pltpu.async_copy(hx_hbm.at[row_v], hxg_v, sem).wait()
        pltpu.async_copy(dinv_hbm.at[row_v], nrm_v, sem).wait()
        pltpu.sync_copy(e_hbm.at[pl.ds(base, CE_)], msg_v)

        def grp(g, cc):
            nv = nrm_v[pl.ds(g * 16, 16)]
            for l in range(16):
                k = g * 16 + l
                ns = nv[l]
                for j in range(D_ // 16):
                    hv = hxg_v[k, pl.ds(j * 16, 16)]
                    ev = msg_v[k, pl.ds(j * 16, 16)]
                    msg_v[k, pl.ds(j * 16, 16)] = jnp.maximum(hv + ev, 0.0) * ns
            return cc

        lax.fori_loop(0, CE_ // 16, grp, 0)
        pltpu.sync_copy(msg_v, agg_sh.at[col_v], add=True)
        return carry

    lax.fori_loop(0, nch, chunk, 0)
    plsc.subcore_barrier()

    @pl.when(s < 15)
    def _():
        pltpu.sync_copy(agg_sh.at[pl.ds(s * RPT_, RPT_)],
                        out_hbm.at[c, pl.ds(s * RPT_, RPT_)])

    @pl.when(s == 15)
    def _():
        pltpu.sync_copy(agg_sh.at[pl.ds(15 * RPT_, RLAST_)],
                        out_hbm.at[c, pl.ds(15 * RPT_, RLAST_)])


@functools.cache
def _sc_edges():
    mesh = plsc.VectorSubcoreMesh(core_axis_name="c", subcore_axis_name="s",
                                  num_cores=NC_, num_subcores=NS_)
    return pl.kernel(
        _edge_body,
        out_type=jax.ShapeDtypeStruct((NC_, N_, D_), jnp.float32),
        mesh=mesh,
        scratch_types=[
            pltpu.VMEM_SHARED((N_, D_), jnp.float32),
            pltpu.VMEM((CE_,), jnp.int32),
            pltpu.VMEM((CE_,), jnp.int32),
            pltpu.VMEM((CE_, D_), jnp.float32),
            pltpu.VMEM((CE_, D_), jnp.float32),
            pltpu.VMEM((CE_,), jnp.float32),
            pltpu.VMEM((8, D_), jnp.float32),
            pltpu.SemaphoreType.DMA,
        ],
    )


def _dense_body(x_ref, wf_ref, bf_ref, vn_ref, wl_ref, bl_ref, degt_ref,
                h0_ref, hx_ref, dinv_ref):
    h0 = jnp.dot(x_ref[...], wf_ref[...], preferred_element_type=jnp.float32)
    h0 = h0 + bf_ref[...] + vn_ref[...]
    h0_ref[...] = h0
    hx_ref[...] = jnp.dot(h0, wl_ref[...],
                          preferred_element_type=jnp.float32) + bl_ref[...]
    deg = degt_ref[:, 0:1] + degt_ref[:, 1:2] + 1.0
    dinv_ref[...] = lax.rsqrt(deg)


def _edge_mlp_body(ea_ref, we_ref, be_ref, e_ref):
    e_ref[...] = jnp.dot(ea_ref[...], we_ref[...],
                         preferred_element_type=jnp.float32) + be_ref[...]


def _final_body(a0_ref, a1_ref, hx_ref, h0_ref, dinv_ref, root_ref,
                g_ref, b_ref, out_ref):
    dinv = dinv_ref[...]
    agg = (a0_ref[...] + a1_ref[...]) * dinv
    h = agg + jnp.maximum(hx_ref[...] + root_ref[...], 0.0) * (dinv * dinv)
    mu = jnp.mean(h, axis=0, keepdims=True)
    xc = h - mu
    var = jnp.mean(xc * xc, axis=0, keepdims=True)
    hn = xc * lax.rsqrt(var + EPS_) * g_ref[...] + b_ref[...]
    out_ref[...] = jnp.maximum(hn, 0.0) + h0_ref[...]


def kernel(x, edge_index, edge_attr, batch, W_feat, b_feat, vn_emb,
           W_lin, b_lin, W_edge, b_edge, root_emb, bn_gamma, bn_beta):
    bf = b_feat.reshape(1, D_)
    bl = b_lin.reshape(1, D_)
    be = b_edge.reshape(1, D_)
    g2 = bn_gamma.reshape(1, D_)
    b2 = bn_beta.reshape(1, D_)

    row = edge_index[0]
    col = edge_index[1]
    cnt = _sc_degree()(row).reshape(NC_, NP_)[:, :N_]
    degt = jnp.transpose(cnt)

    h0, hx, dinv = pl.pallas_call(
        _dense_body,
        out_shape=[
            jax.ShapeDtypeStruct((N_, D_), jnp.float32),
            jax.ShapeDtypeStruct((N_, D_), jnp.float32),
            jax.ShapeDtypeStruct((N_, 1), jnp.float32),
        ],
    )(x, W_feat, bf, vn_emb, W_lin, bl, degt)

    BE = 3200
    e = pl.pallas_call(
        _edge_mlp_body,
        grid=(E_ // BE,),
        in_specs=[
            pl.BlockSpec((BE, DE_), lambda i: (i, 0)),
            pl.BlockSpec((DE_, D_), lambda i: (0, 0)),
            pl.BlockSpec((1, D_), lambda i: (0, 0)),
        ],
        out_specs=pl.BlockSpec((BE, D_), lambda i: (i, 0)),
        out_shape=jax.ShapeDtypeStruct((E_, D_), jnp.float32),
    )(edge_attr, W_edge, be)

    aggp = _sc_edges()(row, col, e, hx, dinv.reshape(N_))

    out = pl.pallas_call(
        _final_body,
        out_shape=jax.ShapeDtypeStruct((N_, D_), jnp.float32),
    )(aggp[0], aggp[1], hx, h0, dinv, root_emb, g2, b2)
    return out


# speedup vs baseline: 8.5310x; 1.1634x over previous
import functools

import jax
import jax.numpy as jnp
from jax import lax
from jax.experimental import pallas as pl
from jax.experimental.pallas import tpu as pltpu
from jax.experimental.pallas import tpu_sc as plsc

N_ = 10000
E_ = 320000
D_ = 128
DE_ = 16
EPS_ = 1e-5

NC_ = 2
NS_ = 16
NW_ = NC_ * NS_
CE_ = 128
NCH_ = E_ // CE_
NCHW_ = (NCH_ + NW_ - 1) // NW_
NP_ = 10112
RPT_ = 632
RLAST_ = N_ - 15 * RPT_
DH_ = D_ // 2


def _deg_body(row_hbm, out_hbm, cnt_sh, idx_v, ones_v, zb_v, sem):
    c = lax.axis_index("c")
    s = lax.axis_index("s")
    w = s * NC_ + c
    nch = jnp.where(w < NCH_ % NW_, NCH_ // NW_ + 1, NCH_ // NW_)

    for i in range(CE_ // 16):
        ones_v[pl.ds(i * 16, 16)] = jnp.ones((16,), jnp.float32)
    for i in range(640 // 16):
        zb_v[pl.ds(i * 16, 16)] = jnp.zeros((16,), jnp.float32)

    @pl.when(s < 15)
    def _():
        pltpu.sync_copy(zb_v, cnt_sh.at[pl.ds(s * 640, 640)])

    @pl.when(s == 15)
    def _():
        pltpu.sync_copy(zb_v.at[pl.ds(0, 512)], cnt_sh.at[pl.ds(9600, 512)])

    plsc.subcore_barrier()

    def chunk(i, carry):
        base = (w + NW_ * i) * CE_
        pltpu.sync_copy(row_hbm.at[pl.ds(base, CE_)], idx_v)
        pltpu.sync_copy(ones_v, cnt_sh.at[idx_v], add=True)
        return carry

    lax.fori_loop(0, nch, chunk, 0)
    plsc.subcore_barrier()

    @pl.when(s < 15)
    def _():
        pltpu.sync_copy(cnt_sh.at[pl.ds(s * 640, 640)],
                        out_hbm.at[pl.ds(c * NP_ + s * 640, 640)])

    @pl.when(s == 15)
    def _():
        pltpu.sync_copy(cnt_sh.at[pl.ds(9600, 512)],
                        out_hbm.at[pl.ds(c * NP_ + 9600, 512)])


@functools.cache
def _sc_degree():
    mesh = plsc.VectorSubcoreMesh(core_axis_name="c", subcore_axis_name="s",
                                  num_cores=NC_, num_subcores=NS_)
    return pl.kernel(
        _deg_body,
        out_type=jax.ShapeDtypeStruct((NC_ * NP_,), jnp.float32),
        mesh=mesh,
        scratch_types=[
            pltpu.VMEM_SHARED((NP_,), jnp.float32),
            pltpu.VMEM((CE_,), jnp.int32),
            pltpu.VMEM((CE_,), jnp.float32),
            pltpu.VMEM((640,), jnp.float32),
            pltpu.SemaphoreType.DMA,
        ],
    )


def _edge_body(rowi_hbm, coli_hbm, e_hbm, hx_hbm, dinv_hbm, zeros_hbm,
               out_hbm,
               agg_sh, row_v, col_v, hxg_v, msg_v, nrm_v, sg):
    c = lax.axis_index("c")
    s = lax.axis_index("s")
    w = s * NC_ + c
    nch = jnp.where(w < NCH_ % NW_, NCH_ // NW_ + 1, NCH_ // NW_)

    @pl.when(s < 15)
    def _():
        pltpu.sync_copy(zeros_hbm, agg_sh.at[pl.ds(s * RPT_, RPT_)])

    @pl.when(s == 15)
    def _():
        pltpu.sync_copy(zeros_hbm.at[pl.ds(0, RLAST_)],
                        agg_sh.at[pl.ds(15 * RPT_, RLAST_)])

    plsc.subcore_barrier()

    def chunk(i, carry):
        base = (w + NW_ * i) * CE_
        pltpu.sync_copy(rowi_hbm.at[pl.ds(base, CE_)], row_v)
        pltpu.async_copy(hx_hbm.at[row_v], hxg_v, sg)
        pltpu.async_copy(dinv_hbm.at[row_v], nrm_v, sg)
        pltpu.sync_copy(coli_hbm.at[pl.ds(base, CE_)], col_v)
        pltpu.sync_copy(e_hbm.at[pl.ds(base, CE_)], msg_v)
        pltpu.make_async_copy(hx_hbm.at[row_v], hxg_v, sg).wait()
        pltpu.make_async_copy(dinv_hbm.at[row_v], nrm_v, sg).wait()

        def grp(g, cc):
            nv = nrm_v[pl.ds(g * 16, 16)]
            for l in range(16):
                k = g * 16 + l
                ns = nv[l]
                for j in range(D_ // 16):
                    hv = hxg_v[k, pl.ds(j * 16, 16)]
                    ev = msg_v[k, pl.ds(j * 16, 16)]
                    msg_v[k, pl.ds(j * 16, 16)] = (
                        jnp.maximum(hv + ev, 0.0) * ns)
            return cc

        lax.fori_loop(0, CE_ // 16, grp, 0)
        pltpu.sync_copy(msg_v, agg_sh.at[col_v], add=True)
        return carry

    lax.fori_loop(0, nch, chunk, 0)
    plsc.subcore_barrier()

    @pl.when(s < 15)
    def _():
        pltpu.sync_copy(agg_sh.at[pl.ds(s * RPT_, RPT_)],
                        out_hbm.at[c, pl.ds(s * RPT_, RPT_)])

    @pl.when(s == 15)
    def _():
        pltpu.sync_copy(agg_sh.at[pl.ds(15 * RPT_, RLAST_)],
                        out_hbm.at[c, pl.ds(15 * RPT_, RLAST_)])


@functools.cache
def _sc_edges():
    mesh = plsc.VectorSubcoreMesh(core_axis_name="c", subcore_axis_name="s",
                                  num_cores=NC_, num_subcores=NS_)
    return pl.kernel(
        _edge_body,
        out_type=jax.ShapeDtypeStruct((NC_, N_, D_), jnp.float32),
        mesh=mesh,
        scratch_types=[
            pltpu.VMEM_SHARED((N_, D_), jnp.float32),
            pltpu.VMEM((CE_,), jnp.int32),
            pltpu.VMEM((CE_,), jnp.int32),
            pltpu.VMEM((CE_, D_), jnp.float32),
            pltpu.VMEM((CE_, D_), jnp.float32),
            pltpu.VMEM((CE_,), jnp.float32),
            pltpu.SemaphoreType.DMA,
        ],
    )


def _dense_body(x_ref, wf_ref, bf_ref, vn_ref, wl_ref, bl_ref, degt_ref,
                h0_ref, hx_ref, dinv_ref):
    h0 = jnp.dot(x_ref[...], wf_ref[...], preferred_element_type=jnp.float32)
    h0 = h0 + bf_ref[...] + vn_ref[...]
    h0_ref[...] = h0
    hx_ref[...] = jnp.dot(h0, wl_ref[...],
                          preferred_element_type=jnp.float32) + bl_ref[...]
    deg = degt_ref[:, 0:1] + degt_ref[:, 1:2] + 1.0
    dinv_ref[...] = lax.rsqrt(deg)


def _edge_mlp_body(ea_ref, we_ref, be_ref, e_ref):
    e_ref[...] = jnp.dot(ea_ref[...], we_ref[...],
                         preferred_element_type=jnp.float32) + be_ref[...]


def _final_body(a0_ref, a1_ref, hx_ref, h0_ref, dinv_ref, root_ref,
                g_ref, b_ref, out_ref):
    dinv = dinv_ref[...]
    agg = (a0_ref[...] + a1_ref[...]) * dinv
    h = agg + jnp.maximum(hx_ref[...] + root_ref[...], 0.0) * (dinv * dinv)
    mu = jnp.mean(h, axis=0, keepdims=True)
    xc = h - mu
    var = jnp.mean(xc * xc, axis=0, keepdims=True)
    hn = xc * lax.rsqrt(var + EPS_) * g_ref[...] + b_ref[...]
    out_ref[...] = jnp.maximum(hn, 0.0) + h0_ref[...]


def kernel(x, edge_index, edge_attr, batch, W_feat, b_feat, vn_emb,
           W_lin, b_lin, W_edge, b_edge, root_emb, bn_gamma, bn_beta):
    bf = b_feat.reshape(1, D_)
    bl = b_lin.reshape(1, D_)
    be = b_edge.reshape(1, D_)
    g2 = bn_gamma.reshape(1, D_)
    b2 = bn_beta.reshape(1, D_)

    row = edge_index[0]
    col = edge_index[1]
    cnt = _sc_degree()(row).reshape(NC_, NP_)[:, :N_]
    degt = jnp.transpose(cnt)

    h0, hx, dinv = pl.pallas_call(
        _dense_body,
        out_shape=[
            jax.ShapeDtypeStruct((N_, D_), jnp.float32),
            jax.ShapeDtypeStruct((N_, D_), jnp.float32),
            jax.ShapeDtypeStruct((N_, 1), jnp.float32),
        ],
    )(x, W_feat, bf, vn_emb, W_lin, bl, degt)

    BE = 3200
    e = pl.pallas_call(
        _edge_mlp_body,
        grid=(E_ // BE,),
        in_specs=[
            pl.BlockSpec((BE, DE_), lambda i: (i, 0)),
            pl.BlockSpec((DE_, D_), lambda i: (0, 0)),
            pl.BlockSpec((1, D_), lambda i: (0, 0)),
        ],
        out_specs=pl.BlockSpec((BE, D_), lambda i: (i, 0)),
        out_shape=jax.ShapeDtypeStruct((E_, D_), jnp.float32),
    )(edge_attr, W_edge, be)

    zeros = jnp.zeros((RPT_, D_), jnp.float32)
    aggp = _sc_edges()(row, col, e, hx, dinv.reshape(N_), zeros)

    out = pl.pallas_call(
        _final_body,
        out_shape=jax.ShapeDtypeStruct((N_, D_), jnp.float32),
    )(aggp[0], aggp[1], hx, h0, dinv, root_emb, g2, b2)
    return out


# speedup vs baseline: 11.5053x; 1.3486x over previous
import functools

import jax
import jax.numpy as jnp
from jax import lax
from jax.experimental import pallas as pl
from jax.experimental.pallas import tpu as pltpu
from jax.experimental.pallas import tpu_sc as plsc

N_ = 10000
E_ = 320000
D_ = 128
DE_ = 16
EPS_ = 1e-5

NC_ = 2
NS_ = 16
NW_ = NC_ * NS_
CE_ = 128
NCH_ = E_ // CE_
NCHW_ = (NCH_ + NW_ - 1) // NW_
NP_ = 10112
RPT_ = 632
RLAST_ = N_ - 15 * RPT_
DH_ = D_ // 2


def _deg_body(row_hbm, out_hbm, cnt_sh, idx_v, ones_v, zb_v, sem):
    c = lax.axis_index("c")
    s = lax.axis_index("s")
    w = s * NC_ + c
    nch = jnp.where(w < NCH_ % NW_, NCH_ // NW_ + 1, NCH_ // NW_)

    for i in range(CE_ // 16):
        ones_v[pl.ds(i * 16, 16)] = jnp.ones((16,), jnp.float32)
    for i in range(640 // 16):
        zb_v[pl.ds(i * 16, 16)] = jnp.zeros((16,), jnp.float32)

    @pl.when(s < 15)
    def _():
        pltpu.sync_copy(zb_v, cnt_sh.at[pl.ds(s * 640, 640)])

    @pl.when(s == 15)
    def _():
        pltpu.sync_copy(zb_v.at[pl.ds(0, 512)], cnt_sh.at[pl.ds(9600, 512)])

    plsc.subcore_barrier()

    def chunk(i, carry):
        base = (w + NW_ * i) * CE_
        pltpu.sync_copy(row_hbm.at[pl.ds(base, CE_)], idx_v)
        pltpu.sync_copy(ones_v, cnt_sh.at[idx_v], add=True)
        return carry

    lax.fori_loop(0, nch, chunk, 0)
    plsc.subcore_barrier()

    @pl.when(s < 15)
    def _():
        pltpu.sync_copy(cnt_sh.at[pl.ds(s * 640, 640)],
                        out_hbm.at[pl.ds(c * NP_ + s * 640, 640)])

    @pl.when(s == 15)
    def _():
        pltpu.sync_copy(cnt_sh.at[pl.ds(9600, 512)],
                        out_hbm.at[pl.ds(c * NP_ + 9600, 512)])


@functools.cache
def _sc_degree():
    mesh = plsc.VectorSubcoreMesh(core_axis_name="c", subcore_axis_name="s",
                                  num_cores=NC_, num_subcores=NS_)
    return pl.kernel(
        _deg_body,
        out_type=jax.ShapeDtypeStruct((NC_ * NP_,), jnp.float32),
        mesh=mesh,
        scratch_types=[
            pltpu.VMEM_SHARED((NP_,), jnp.float32),
            pltpu.VMEM((CE_,), jnp.int32),
            pltpu.VMEM((CE_,), jnp.float32),
            pltpu.VMEM((640,), jnp.float32),
            pltpu.SemaphoreType.DMA,
        ],
    )


def _edge_body(rowi_hbm, coli_hbm, e_hbm, hx_hbm, dinv_hbm, zeros_hbm,
               out_hbm,
               agg_sh, row_v, col_v, cols_v, hxg_v, msg_v, nrm_v, sg, ss):
    c = lax.axis_index("c")
    s = lax.axis_index("s")
    w = s * NC_ + c
    nch = jnp.where(w < NCH_ % NW_, NCH_ // NW_ + 1, NCH_ // NW_)

    @pl.when(s < 15)
    def _():
        pltpu.sync_copy(zeros_hbm, agg_sh.at[pl.ds(s * RPT_, RPT_)])

    @pl.when(s == 15)
    def _():
        pltpu.sync_copy(zeros_hbm.at[pl.ds(0, RLAST_)],
                        agg_sh.at[pl.ds(15 * RPT_, RLAST_)])

    plsc.subcore_barrier()

    def chunk(i, carry):
        base = (w + NW_ * i) * CE_
        pltpu.sync_copy(rowi_hbm.at[pl.ds(base, CE_)], row_v)
        pltpu.async_copy(hx_hbm.at[row_v], hxg_v, sg)
        pltpu.async_copy(dinv_hbm.at[row_v], nrm_v, sg)
        pltpu.sync_copy(coli_hbm.at[pl.ds(base, CE_)], col_v)

        @pl.when(i >= 1)
        def _():
            pltpu.make_async_copy(msg_v, agg_sh.at[cols_v], ss).wait()

        pltpu.sync_copy(e_hbm.at[pl.ds(base, CE_)], msg_v)
        pltpu.make_async_copy(hx_hbm.at[row_v], hxg_v, sg).wait()
        pltpu.make_async_copy(dinv_hbm.at[row_v], nrm_v, sg).wait()

        def grp(g, cc):
            nv = nrm_v[pl.ds(g * 16, 16)]
            cols_v[pl.ds(g * 16, 16)] = col_v[pl.ds(g * 16, 16)]
            for l in range(16):
                k = g * 16 + l
                ns = nv[l]
                for j in range(D_ // 16):
                    hv = hxg_v[k, pl.ds(j * 16, 16)]
                    ev = msg_v[k, pl.ds(j * 16, 16)]
                    msg_v[k, pl.ds(j * 16, 16)] = (
                        jnp.maximum(hv + ev, 0.0) * ns)
            return cc

        lax.fori_loop(0, CE_ // 16, grp, 0)
        pltpu.async_copy(msg_v, agg_sh.at[cols_v], ss, add=True)
        return carry

    lax.fori_loop(0, nch, chunk, 0)
    pltpu.make_async_copy(msg_v, agg_sh.at[cols_v], ss).wait()
    plsc.subcore_barrier()

    @pl.when(s < 15)
    def _():
        pltpu.sync_copy(agg_sh.at[pl.ds(s * RPT_, RPT_)],
                        out_hbm.at[c, pl.ds(s * RPT_, RPT_)])

    @pl.when(s == 15)
    def _():
        pltpu.sync_copy(agg_sh.at[pl.ds(15 * RPT_, RLAST_)],
                        out_hbm.at[c, pl.ds(15 * RPT_, RLAST_)])


@functools.cache
def _sc_edges():
    mesh = plsc.VectorSubcoreMesh(core_axis_name="c", subcore_axis_name="s",
                                  num_cores=NC_, num_subcores=NS_)
    return pl.kernel(
        _edge_body,
        out_type=jax.ShapeDtypeStruct((NC_, N_, D_), jnp.float32),
        mesh=mesh,
        scratch_types=[
            pltpu.VMEM_SHARED((N_, D_), jnp.float32),
            pltpu.VMEM((CE_,), jnp.int32),
            pltpu.VMEM((CE_,), jnp.int32),
            pltpu.VMEM((CE_,), jnp.int32),
            pltpu.VMEM((CE_, D_), jnp.float32),
            pltpu.VMEM((CE_, D_), jnp.float32),
            pltpu.VMEM((CE_,), jnp.float32),
            pltpu.SemaphoreType.DMA,
            pltpu.SemaphoreType.DMA,
        ],
    )


def _dense_body(x_ref, wf_ref, bf_ref, vn_ref, wl_ref, bl_ref, degt_ref,
                h0_ref, hx_ref, dinv_ref):
    h0 = jnp.dot(x_ref[...], wf_ref[...], preferred_element_type=jnp.float32)
    h0 = h0 + bf_ref[...] + vn_ref[...]
    h0_ref[...] = h0
    hx_ref[...] = jnp.dot(h0, wl_ref[...],
                          preferred_element_type=jnp.float32) + bl_ref[...]
    deg = degt_ref[:, 0:1] + degt_ref[:, 1:2] + 1.0
    dinv_ref[...] = lax.rsqrt(deg)


def _edge_mlp_body(ea_ref, we_ref, be_ref, e_ref):
    e_ref[...] = jnp.dot(ea_ref[...], we_ref[...],
                         preferred_element_type=jnp.float32) + be_ref[...]


def _final_body(a0_ref, a1_ref, hx_ref, h0_ref, dinv_ref, root_ref,
                g_ref, b_ref, out_ref):
    dinv = dinv_ref[...]
    agg = (a0_ref[...] + a1_ref[...]) * dinv
    h = agg + jnp.maximum(hx_ref[...] + root_ref[...], 0.0) * (dinv * dinv)
    mu = jnp.mean(h, axis=0, keepdims=True)
    xc = h - mu
    var = jnp.mean(xc * xc, axis=0, keepdims=True)
    hn = xc * lax.rsqrt(var + EPS_) * g_ref[...] + b_ref[...]
    out_ref[...] = jnp.maximum(hn, 0.0) + h0_ref[...]


def kernel(x, edge_index, edge_attr, batch, W_feat, b_feat, vn_emb,
           W_lin, b_lin, W_edge, b_edge, root_emb, bn_gamma, bn_beta):
    bf = b_feat.reshape(1, D_)
    bl = b_lin.reshape(1, D_)
    be = b_edge.reshape(1, D_)
    g2 = bn_gamma.reshape(1, D_)
    b2 = bn_beta.reshape(1, D_)

    row = edge_index[0]
    col = edge_index[1]
    cnt = _sc_degree()(row).reshape(NC_, NP_)[:, :N_]
    degt = jnp.transpose(cnt)

    h0, hx, dinv = pl.pallas_call(
        _dense_body,
        out_shape=[
            jax.ShapeDtypeStruct((N_, D_), jnp.float32),
            jax.ShapeDtypeStruct((N_, D_), jnp.float32),
            jax.ShapeDtypeStruct((N_, 1), jnp.float32),
        ],
    )(x, W_feat, bf, vn_emb, W_lin, bl, degt)

    BE = 3200
    e = pl.pallas_call(
        _edge_mlp_body,
        grid=(E_ // BE,),
        in_specs=[
            pl.BlockSpec((BE, DE_), lambda i: (i, 0)),
            pl.BlockSpec((DE_, D_), lambda i: (0, 0)),
            pl.BlockSpec((1, D_), lambda i: (0, 0)),
        ],
        out_specs=pl.BlockSpec((BE, D_), lambda i: (i, 0)),
        out_shape=jax.ShapeDtypeStruct((E_, D_), jnp.float32),
    )(edge_attr, W_edge, be)

    zeros = jnp.zeros((RPT_, D_), jnp.float32)
    aggp = _sc_edges()(row, col, e, hx, dinv.reshape(N_), zeros)

    out = pl.pallas_call(
        _final_body,
        out_shape=jax.ShapeDtypeStruct((N_, D_), jnp.float32),
    )(aggp[0], aggp[1], hx, h0, dinv, root_emb, g2, b2)
    return out
